# baseline (speedup 1.0000x reference)
import functools

import jax
import jax.numpy as jnp
from jax import lax
from jax.experimental import pallas as pl
from jax.experimental.pallas import tpu as pltpu
from jax.experimental.pallas import tpu_sc as plsc

N = 49995
E = 799920
D_IN = 128
D_H = 64

N_PAD = 50176
ROWS_PER_TILE = N_PAD // 16

K = 432
CH = 116
E_PAD = 16 * K * CH
DCH = E_PAD // (32 * K)

f32 = jnp.float32


def _sc_degree_body(dst_hbm, ones_hbm, zeros_hbm, out_hbm,
                    idx_v0, idx_v1, idx_v2, ones_v, deg_sh,
                    is0, is1, is2, ss0, ss1, ss2):
    c = lax.axis_index("c")
    s = lax.axis_index("s")
    pltpu.sync_copy(ones_hbm, ones_v)
    r0 = s * ROWS_PER_TILE
    pltpu.sync_copy(zeros_hbm.at[pl.ds(r0, ROWS_PER_TILE)],
                    deg_sh.at[pl.ds(r0, ROWS_PER_TILE)])
    plsc.subcore_barrier()
    base = (c * 16 + s) * DCH * K
    bufs = ((idx_v0, is0, ss0), (idx_v1, is1, ss1), (idx_v2, is2, ss2))

    def stage_idx(i, t):
        iv, isem, _ = bufs[t]
        pltpu.async_copy(dst_hbm.at[pl.ds(base + i * K, K)], iv, isem)

    def scatter(i, t):
        iv, isem, ss = bufs[t]
        pltpu.make_async_copy(dst_hbm.at[pl.ds(base + i * K, K)], iv,
                              isem).wait()
        pltpu.async_copy(ones_v, deg_sh.at[iv], ss, add=True)

    def wait_scatter(t):
        iv, _, ss = bufs[t]
        pltpu.make_async_copy(ones_v, deg_sh.at[iv], ss).wait()

    stage_idx(0, 0)
    stage_idx(1, 1)
    stage_idx(2, 2)
    scatter(0, 0)
    scatter(1, 1)

    peel = 3 + (DCH - 2) % 3
    loop_n = (DCH - 2 - peel) // 3

    @pl.loop(0, loop_n)
    def _outer(o):
        for u in range(3):
            i = 2 + 3 * o + u
            t = (2 + u) % 3
            wait_scatter(u % 3)
            stage_idx(i + 1, u % 3)
            scatter(i, t)

    for u in range(peel):
        i = 2 + 3 * loop_n + u
        t = (2 + u) % 3
        wait_scatter(u % 3)
        if i + 1 < DCH:
            stage_idx(i + 1, u % 3)
        scatter(i, t)

    wait_scatter((DCH - 2) % 3)
    wait_scatter((DCH - 1) % 3)
    plsc.subcore_barrier()
    pltpu.sync_copy(deg_sh.at[pl.ds(r0, ROWS_PER_TILE)],
                    out_hbm.at[c, pl.ds(r0, ROWS_PER_TILE)])


def _make_sc_scatter_body():
    def body(tlo_hbm, thi_hbm, src_hbm, dst_hbm, zeros_hbm, out_hbm,
             src_v0, dst_v0, src_v1, dst_v1, src_v2, dst_v2,
             rows_v0, rows_v1, agg_sh,
             is0, is1, is2, gs0, gs1, ss0, ss1):
        c = lax.axis_index("c")
        s = lax.axis_index("s")
        r0 = s * ROWS_PER_TILE
        pltpu.sync_copy(zeros_hbm.at[pl.ds(r0, ROWS_PER_TILE)],
                        agg_sh.at[pl.ds(r0, ROWS_PER_TILE)])
        plsc.subcore_barrier()
        base = s * CH * K
        ibufs = ((src_v0, dst_v0, is0), (src_v1, dst_v1, is1),
                 (src_v2, dst_v2, is2))
        rbufs = ((rows_v0, gs0, ss0), (rows_v1, gs1, ss1))

        def stage_idx(i, t):
            sv, dv, isem = ibufs[t]
            pltpu.async_copy(src_hbm.at[pl.ds(base + i * K, K)], sv, isem)
            pltpu.async_copy(dst_hbm.at[pl.ds(base + i * K, K)], dv, isem)

        def wait_idx(i, t):
            sv, dv, isem = ibufs[t]
            pltpu.make_async_copy(src_hbm.at[pl.ds(base + i * K, K)], sv,
                                  isem).wait()
            pltpu.make_async_copy(dst_hbm.at[pl.ds(base + i * K, K)], dv,
                                  isem).wait()

        def run(tab):
            def gather_scatter(i, t, b):
                sv, dv, _ = ibufs[t]
                rv, gs, ss = rbufs[b]
                wait_idx(i, t)
                pltpu.async_copy(tab.at[sv], rv, gs)
                pltpu.make_async_copy(tab.at[sv], rv, gs).wait()
                pltpu.async_copy(rv, agg_sh.at[dv], ss, add=True)

            def wait_scatter(b, t_idx):
                rv, gs, ss = rbufs[b]
                pltpu.make_async_copy(rv, agg_sh.at[ibufs[t_idx][1]],
                                      ss).wait()

            stage_idx(0, 0)
            stage_idx(1, 1)
            stage_idx(2, 2)
            gather_scatter(0, 0, 0)
            gather_scatter(1, 1, 1)

            @pl.loop(0, (CH - 8) // 6)
            def _outer(o):
                for u in range(6):
                    i = 2 + 6 * o + u
                    b = u % 2
                    t = (2 + u) % 3
                    wait_scatter(b, u % 3)
                    stage_idx(i + 1, u % 3)
                    gather_scatter(i, t, b)

            for u in range(6):
                i = CH - 6 + u
                b = i % 2
                t = i % 3
                wait_scatter(b, (i - 2) % 3)
                if i + 1 < CH:
                    stage_idx(i + 1, (i + 1) % 3)
                gather_scatter(i, t, b)

            wait_scatter(0, (CH - 2) % 3)
            wait_scatter(1, (CH - 1) % 3)

        @pl.when(c == 0)
        def _lo():
            run(tlo_hbm)

        @pl.when(c == 1)
        def _hi():
            run(thi_hbm)

        plsc.subcore_barrier()
        pltpu.sync_copy(agg_sh.at[pl.ds(r0, ROWS_PER_TILE)],
                        out_hbm.at[c, pl.ds(r0, ROWS_PER_TILE)])

    return body


@functools.cache
def _sc_kernels():
    mesh = plsc.VectorSubcoreMesh(core_axis_name="c", subcore_axis_name="s")
    params = pltpu.CompilerParams(use_tc_tiling_on_sc=False)
    sc_degree = pl.kernel(
        _sc_degree_body,
        out_type=jax.ShapeDtypeStruct((2, N_PAD, 32), f32),
        mesh=mesh,
        compiler_params=params,
        scratch_types=(
            [pltpu.VMEM((K,), jnp.int32)] * 3
            + [pltpu.VMEM((K, 32), f32)]
            + [pltpu.VMEM_SHARED((N_PAD, 32), f32)]
            + [pltpu.SemaphoreType.DMA] * 6
        ),
    )
    sc_scatter = pl.kernel(
        _make_sc_scatter_body(),
        out_type=jax.ShapeDtypeStruct((2, N_PAD, 32), f32),
        mesh=mesh,
        compiler_params=params,
        scratch_types=(
            [pltpu.VMEM((K,), jnp.int32)] * 6
            + [pltpu.VMEM((K, 32), f32)] * 2
            + [pltpu.VMEM_SHARED((N_PAD, 32), f32)]
            + [pltpu.SemaphoreType.DMA] * 7
        ),
    )
    return sc_degree, sc_scatter


RB = 512
RP = RB // 4
GRID = N_PAD // RB


def _mm1_body(x_ref, wlo_ref, whi_ref, hlo_ref, hhi_ref):
    x4 = x_ref[...]
    hlo_ref[...] = jnp.dot(x4, wlo_ref[...], preferred_element_type=f32)
    hhi_ref[...] = jnp.dot(x4, whi_ref[...], preferred_element_type=f32)


_mm1 = pl.pallas_call(
    _mm1_body,
    grid=(GRID,),
    in_specs=[
        pl.BlockSpec((RP, 4 * D_IN), lambda i: (i, 0)),
        pl.BlockSpec((4 * D_IN, 128), lambda i: (0, 0)),
        pl.BlockSpec((4 * D_IN, 128), lambda i: (0, 0)),
    ],
    out_specs=[
        pl.BlockSpec((RP, 128), lambda i: (i, 0)),
        pl.BlockSpec((RP, 128), lambda i: (i, 0)),
    ],
    out_shape=[
        jax.ShapeDtypeStruct((N_PAD // 4, 128), f32),
        jax.ShapeDtypeStruct((N_PAD // 4, 128), f32),
    ],
)


def _scale1_body(d_ref, hlo_ref, hhi_ref, tlo_ref, thi_ref, dinv_ref):
    dinv = lax.rsqrt(d_ref[0] + d_ref[1] + 1.0)
    tlo_ref[...] = dinv * hlo_ref[...]
    thi_ref[...] = dinv * hhi_ref[...]
    dinv_ref[...] = dinv


_scale1 = pl.pallas_call(
    _scale1_body,
    grid=(GRID,),
    in_specs=[
        pl.BlockSpec((2, RP, 128), lambda i: (0, i, 0)),
        pl.BlockSpec((RP, 128), lambda i: (i, 0)),
        pl.BlockSpec((RP, 128), lambda i: (i, 0)),
    ],
    out_specs=[
        pl.BlockSpec((RP, 128), lambda i: (i, 0)),
        pl.BlockSpec((RP, 128), lambda i: (i, 0)),
        pl.BlockSpec((RP, 128), lambda i: (i, 0)),
    ],
    out_shape=[
        jax.ShapeDtypeStruct((N_PAD // 4, 128), f32),
        jax.ShapeDtypeStruct((N_PAD // 4, 128), f32),
        jax.ShapeDtypeStruct((N_PAD // 4, 128), f32),
    ],
)


def _mid_body(agg_ref, tlo_ref, thi_ref, dinv_ref,
              waa_ref, wba_ref, wab_ref, wbb_ref, blo_ref, bhi_ref,
              olo_ref, ohi_ref):
    dinv = dinv_ref[...]
    x2lo = jax.nn.relu(dinv * (agg_ref[0] + tlo_ref[...]) + blo_ref[...])
    x2hi = jax.nn.relu(dinv * (agg_ref[1] + thi_ref[...]) + bhi_ref[...])
    h2lo = (jnp.dot(x2lo, waa_ref[...], preferred_element_type=f32)
            + jnp.dot(x2hi, wba_ref[...], preferred_element_type=f32))
    h2hi = (jnp.dot(x2lo, wab_ref[...], preferred_element_type=f32)
            + jnp.dot(x2hi, wbb_ref[...], preferred_element_type=f32))
    olo_ref[...] = dinv * h2lo
    ohi_ref[...] = dinv * h2hi


_mid = pl.pallas_call(
    _mid_body,
    grid=(GRID,),
    in_specs=[
        pl.BlockSpec((2, RP, 128), lambda i: (0, i, 0)),
        pl.BlockSpec((RP, 128), lambda i: (i, 0)),
        pl.BlockSpec((RP, 128), lambda i: (i, 0)),
        pl.BlockSpec((RP, 128), lambda i: (i, 0)),
        pl.BlockSpec((128, 128), lambda i: (0, 0)),
        pl.BlockSpec((128, 128), lambda i: (0, 0)),
        pl.BlockSpec((128, 128), lambda i: (0, 0)),
        pl.BlockSpec((128, 128), lambda i: (0, 0)),
        pl.BlockSpec((1, 128), lambda i: (0, 0)),
        pl.BlockSpec((1, 128), lambda i: (0, 0)),
    ],
    out_specs=[
        pl.BlockSpec((RP, 128), lambda i: (i, 0)),
        pl.BlockSpec((RP, 128), lambda i: (i, 0)),
    ],
    out_shape=[
        jax.ShapeDtypeStruct((N_PAD // 4, 128), f32),
        jax.ShapeDtypeStruct((N_PAD // 4, 128), f32),
    ],
)


def _head_body(agg_ref, tlo_ref, thi_ref, dinv_ref,
               w3lo_ref, w3hi_ref, blo_ref, bhi_ref, s_ref, b3_ref, out_ref):
    dinv = dinv_ref[...]
    x3lo = jax.nn.relu(dinv * (agg_ref[0] + tlo_ref[...]) + blo_ref[...])
    x3hi = jax.nn.relu(dinv * (agg_ref[1] + thi_ref[...]) + bhi_ref[...])
    z = x3lo * w3lo_ref[...] + x3hi * w3hi_ref[...]
    out_ref[...] = (jnp.dot(z, s_ref[...], preferred_element_type=f32)
                    + b3_ref[0, 0])


_head = pl.pallas_call(
    _head_body,
    grid=(GRID,),
    in_specs=[
        pl.BlockSpec((2, RP, 128), lambda i: (0, i, 0)),
        pl.BlockSpec((RP, 128), lambda i: (i, 0)),
        pl.BlockSpec((RP, 128), lambda i: (i, 0)),
        pl.BlockSpec((RP, 128), lambda i: (i, 0)),
        pl.BlockSpec((1, 128), lambda i: (0, 0)),
        pl.BlockSpec((1, 128), lambda i: (0, 0)),
        pl.BlockSpec((1, 128), lambda i: (0, 0)),
        pl.BlockSpec((1, 128), lambda i: (0, 0)),
        pl.BlockSpec((128, 4), lambda i: (0, 0)),
        pl.BlockSpec((1, 8), lambda i: (0, 0)),
    ],
    out_specs=pl.BlockSpec((RP, 4), lambda i: (i, 0)),
    out_shape=jax.ShapeDtypeStruct((N_PAD // 4, 4), f32),
)


def kernel(obs, edge_index, W1, b1, W2, b2, W3, b3):
    src = edge_index[0]
    dst = edge_index[1]
    pad = E_PAD - E
    ar = jnp.arange(pad, dtype=jnp.int32)
    src_p = jnp.concatenate([src, ar % N])
    dst_p = jnp.concatenate([dst, N + ar % (N_PAD - N)])

    obs4 = jnp.pad(obs, ((0, N_PAD - N), (0, 0))).reshape(N_PAD // 4, 4 * D_IN)
    zeros32 = jnp.zeros((N_PAD, 32), f32)
    ones32 = jnp.ones((K, 32), f32)

    eye4 = jnp.eye(4, dtype=f32)
    w1lo = jnp.kron(eye4, W1[:, :32])
    w1hi = jnp.kron(eye4, W1[:, 32:])
    w2aa = jnp.kron(eye4, W2[:32, :32])
    w2ba = jnp.kron(eye4, W2[32:, :32])
    w2ab = jnp.kron(eye4, W2[:32, 32:])
    w2bb = jnp.kron(eye4, W2[32:, 32:])
    b1lo = jnp.tile(b1[:32], 4).reshape(1, 128)
    b1hi = jnp.tile(b1[32:], 4).reshape(1, 128)
    b2lo = jnp.tile(b2[:32], 4).reshape(1, 128)
    b2hi = jnp.tile(b2[32:], 4).reshape(1, 128)
    w3lo = jnp.tile(W3[:32, 0], 4).reshape(1, 128)
    w3hi = jnp.tile(W3[32:, 0], 4).reshape(1, 128)
    ssum = jnp.kron(eye4, jnp.ones((32, 1), f32))
    b3b = jnp.broadcast_to(b3.reshape(1, 1), (1, 8))

    _sc_degree, _sc_scatter = _sc_kernels()
    degp = _sc_degree(dst_p, ones32, zeros32)

    h1lo, h1hi = _mm1(obs4, w1lo, w1hi)
    tab1lo, tab1hi, dinv = _scale1(degp.reshape(2, N_PAD // 4, 128),
                                   h1lo, h1hi)
    agg1 = _sc_scatter(tab1lo.reshape(N_PAD, 32), tab1hi.reshape(N_PAD, 32),
                       src_p, dst_p, zeros32)

    tab2lo, tab2hi = _mid(agg1.reshape(2, N_PAD // 4, 128), tab1lo, tab1hi,
                          dinv, w2aa, w2ba, w2ab, w2bb, b1lo, b1hi)
    agg2 = _sc_scatter(tab2lo.reshape(N_PAD, 32), tab2hi.reshape(N_PAD, 32),
                       src_p, dst_p, zeros32)

    y4 = _head(agg2.reshape(2, N_PAD // 4, 128), tab2lo, tab2hi, dinv,
               w3lo, w3hi, b2lo, b2hi, ssum, b3b)

    y = y4.reshape(-1)[:N]
    return y.reshape(-1, 15)[:, 3:].reshape(-1)


# speedup vs baseline: 1.0644x; 1.0644x over previous
import functools

import jax
import jax.numpy as jnp
from jax import lax
from jax.experimental import pallas as pl
from jax.experimental.pallas import tpu as pltpu
from jax.experimental.pallas import tpu_sc as plsc

N = 49995
E = 799920
D_IN = 128
D_H = 64

N_PAD = 50176
ROWS_PER_TILE = N_PAD // 16

K = 432
CH = 116
E_PAD = 16 * K * CH
DCH = E_PAD // (32 * K)

f32 = jnp.float32


def _sc_degree_body(dst_hbm, ones_hbm, zeros_hbm, out_hbm,
                    idx_v0, idx_v1, idx_v2, ones_v, deg_sh,
                    is0, is1, is2, ss0, ss1, ss2):
    c = lax.axis_index("c")
    s = lax.axis_index("s")
    pltpu.sync_copy(ones_hbm, ones_v)
    r0 = s * ROWS_PER_TILE
    pltpu.sync_copy(zeros_hbm.at[pl.ds(r0, ROWS_PER_TILE)],
                    deg_sh.at[pl.ds(r0, ROWS_PER_TILE)])
    plsc.subcore_barrier()
    base = (c * 16 + s) * DCH * K
    bufs = ((idx_v0, is0, ss0), (idx_v1, is1, ss1), (idx_v2, is2, ss2))

    def stage_idx(i, t):
        iv, isem, _ = bufs[t]
        pltpu.async_copy(dst_hbm.at[pl.ds(base + i * K, K)], iv, isem)

    def scatter(i, t):
        iv, isem, ss = bufs[t]
        pltpu.make_async_copy(dst_hbm.at[pl.ds(base + i * K, K)], iv,
                              isem).wait()
        pltpu.async_copy(ones_v, deg_sh.at[iv], ss, add=True)

    def wait_scatter(t):
        iv, _, ss = bufs[t]
        pltpu.make_async_copy(ones_v, deg_sh.at[iv], ss).wait()

    stage_idx(0, 0)
    stage_idx(1, 1)
    stage_idx(2, 2)
    scatter(0, 0)
    scatter(1, 1)

    peel = 3 + (DCH - 2) % 3
    loop_n = (DCH - 2 - peel) // 3

    @pl.loop(0, loop_n)
    def _outer(o):
        for u in range(3):
            i = 2 + 3 * o + u
            t = (2 + u) % 3
            wait_scatter(u % 3)
            stage_idx(i + 1, u % 3)
            scatter(i, t)

    for u in range(peel):
        i = 2 + 3 * loop_n + u
        t = (2 + u) % 3
        wait_scatter(u % 3)
        if i + 1 < DCH:
            stage_idx(i + 1, u % 3)
        scatter(i, t)

    wait_scatter((DCH - 2) % 3)
    wait_scatter((DCH - 1) % 3)
    plsc.subcore_barrier()
    pltpu.sync_copy(deg_sh.at[pl.ds(r0, ROWS_PER_TILE)],
                    out_hbm.at[c, pl.ds(r0, ROWS_PER_TILE)])


def _make_sc_scatter_body():
    def body(tlo_hbm, thi_hbm, src_hbm, dst_hbm, zeros_hbm, out_hbm,
             src_v0, dst_v0, src_v1, dst_v1, src_v2, dst_v2,
             rows_v0, rows_v1, agg_sh,
             is0, is1, is2, gs0, gs1, ss0, ss1):
        c = lax.axis_index("c")
        s = lax.axis_index("s")
        r0 = s * ROWS_PER_TILE
        pltpu.sync_copy(zeros_hbm.at[pl.ds(r0, ROWS_PER_TILE)],
                        agg_sh.at[pl.ds(r0, ROWS_PER_TILE)])
        plsc.subcore_barrier()
        base = s * CH * K
        ibufs = ((src_v0, dst_v0, is0), (src_v1, dst_v1, is1),
                 (src_v2, dst_v2, is2))
        rbufs = ((rows_v0, gs0, ss0), (rows_v1, gs1, ss1))

        def stage_idx(i, t):
            sv, dv, isem = ibufs[t]
            pltpu.async_copy(src_hbm.at[pl.ds(base + i * K, K)], sv, isem)
            pltpu.async_copy(dst_hbm.at[pl.ds(base + i * K, K)], dv, isem)

        def wait_idx(i, t):
            sv, dv, isem = ibufs[t]
            pltpu.make_async_copy(src_hbm.at[pl.ds(base + i * K, K)], sv,
                                  isem).wait()
            pltpu.make_async_copy(dst_hbm.at[pl.ds(base + i * K, K)], dv,
                                  isem).wait()

        def run(tab):
            def gather_scatter(i, t, b):
                sv, dv, _ = ibufs[t]
                rv, gs, ss = rbufs[b]
                wait_idx(i, t)
                pltpu.async_copy(tab.at[sv], rv, gs)
                pltpu.make_async_copy(tab.at[sv], rv, gs).wait()
                pltpu.async_copy(rv, agg_sh.at[dv], ss, add=True)

            def wait_scatter(b, t_idx):
                rv, gs, ss = rbufs[b]
                pltpu.make_async_copy(rv, agg_sh.at[ibufs[t_idx][1]],
                                      ss).wait()

            stage_idx(0, 0)
            stage_idx(1, 1)
            stage_idx(2, 2)
            gather_scatter(0, 0, 0)
            gather_scatter(1, 1, 1)

            @pl.loop(0, (CH - 8) // 6)
            def _outer(o):
                for u in range(6):
                    i = 2 + 6 * o + u
                    b = u % 2
                    t = (2 + u) % 3
                    wait_scatter(b, u % 3)
                    stage_idx(i + 1, u % 3)
                    gather_scatter(i, t, b)

            for u in range(6):
                i = CH - 6 + u
                b = i % 2
                t = i % 3
                wait_scatter(b, (i - 2) % 3)
                if i + 1 < CH:
                    stage_idx(i + 1, (i + 1) % 3)
                gather_scatter(i, t, b)

            wait_scatter(0, (CH - 2) % 3)
            wait_scatter(1, (CH - 1) % 3)

        @pl.when(c == 0)
        def _lo():
            run(tlo_hbm)

        @pl.when(c == 1)
        def _hi():
            run(thi_hbm)

        plsc.subcore_barrier()
        pltpu.sync_copy(agg_sh.at[pl.ds(r0, ROWS_PER_TILE)],
                        out_hbm.at[c, pl.ds(r0, ROWS_PER_TILE)])

    return body


@functools.cache
def _sc_kernels():
    mesh = plsc.VectorSubcoreMesh(core_axis_name="c", subcore_axis_name="s")
    params = pltpu.CompilerParams(use_tc_tiling_on_sc=False)
    sc_degree = pl.kernel(
        _sc_degree_body,
        out_type=jax.ShapeDtypeStruct((2, N_PAD, 32), f32),
        mesh=mesh,
        compiler_params=params,
        scratch_types=(
            [pltpu.VMEM((K,), jnp.int32)] * 3
            + [pltpu.VMEM((K, 32), f32)]
            + [pltpu.VMEM_SHARED((N_PAD, 32), f32)]
            + [pltpu.SemaphoreType.DMA] * 6
        ),
    )
    sc_scatter = pl.kernel(
        _make_sc_scatter_body(),
        out_type=jax.ShapeDtypeStruct((2, N_PAD, 32), f32),
        mesh=mesh,
        compiler_params=params,
        scratch_types=(
            [pltpu.VMEM((K,), jnp.int32)] * 6
            + [pltpu.VMEM((K, 32), f32)] * 2
            + [pltpu.VMEM_SHARED((N_PAD, 32), f32)]
            + [pltpu.SemaphoreType.DMA] * 7
        ),
    )
    return sc_degree, sc_scatter


RB = 512
RP = RB // 4
GRID = N_PAD // RB


def _scale1_body(d_ref, x_ref, wlo_ref, whi_ref, tlo_ref, thi_ref, dinv_ref):
    dinv = lax.rsqrt(d_ref[0] + d_ref[1] + 1.0)
    x4 = x_ref[...]
    tlo_ref[...] = dinv * jnp.dot(x4, wlo_ref[...], preferred_element_type=f32)
    thi_ref[...] = dinv * jnp.dot(x4, whi_ref[...], preferred_element_type=f32)
    dinv_ref[...] = dinv


_scale1 = pl.pallas_call(
    _scale1_body,
    grid=(GRID,),
    in_specs=[
        pl.BlockSpec((2, RP, 128), lambda i: (0, i, 0)),
        pl.BlockSpec((RP, 4 * D_IN), lambda i: (i, 0)),
        pl.BlockSpec((4 * D_IN, 128), lambda i: (0, 0)),
        pl.BlockSpec((4 * D_IN, 128), lambda i: (0, 0)),
    ],
    out_specs=[
        pl.BlockSpec((RP, 128), lambda i: (i, 0)),
        pl.BlockSpec((RP, 128), lambda i: (i, 0)),
        pl.BlockSpec((RP, 128), lambda i: (i, 0)),
    ],
    out_shape=[
        jax.ShapeDtypeStruct((N_PAD // 4, 128), f32),
        jax.ShapeDtypeStruct((N_PAD // 4, 128), f32),
        jax.ShapeDtypeStruct((N_PAD // 4, 128), f32),
    ],
)


def _mid_body(agg_ref, tlo_ref, thi_ref, dinv_ref,
              waa_ref, wba_ref, wab_ref, wbb_ref, blo_ref, bhi_ref,
              olo_ref, ohi_ref):
    dinv = dinv_ref[...]
    x2lo = jax.nn.relu(dinv * (agg_ref[0] + tlo_ref[...]) + blo_ref[...])
    x2hi = jax.nn.relu(dinv * (agg_ref[1] + thi_ref[...]) + bhi_ref[...])
    h2lo = (jnp.dot(x2lo, waa_ref[...], preferred_element_type=f32)
            + jnp.dot(x2hi, wba_ref[...], preferred_element_type=f32))
    h2hi = (jnp.dot(x2lo, wab_ref[...], preferred_element_type=f32)
            + jnp.dot(x2hi, wbb_ref[...], preferred_element_type=f32))
    olo_ref[...] = dinv * h2lo
    ohi_ref[...] = dinv * h2hi


_mid = pl.pallas_call(
    _mid_body,
    grid=(GRID,),
    in_specs=[
        pl.BlockSpec((2, RP, 128), lambda i: (0, i, 0)),
        pl.BlockSpec((RP, 128), lambda i: (i, 0)),
        pl.BlockSpec((RP, 128), lambda i: (i, 0)),
        pl.BlockSpec((RP, 128), lambda i: (i, 0)),
        pl.BlockSpec((128, 128), lambda i: (0, 0)),
        pl.BlockSpec((128, 128), lambda i: (0, 0)),
        pl.BlockSpec((128, 128), lambda i: (0, 0)),
        pl.BlockSpec((128, 128), lambda i: (0, 0)),
        pl.BlockSpec((1, 128), lambda i: (0, 0)),
        pl.BlockSpec((1, 128), lambda i: (0, 0)),
    ],
    out_specs=[
        pl.BlockSpec((RP, 128), lambda i: (i, 0)),
        pl.BlockSpec((RP, 128), lambda i: (i, 0)),
    ],
    out_shape=[
        jax.ShapeDtypeStruct((N_PAD // 4, 128), f32),
        jax.ShapeDtypeStruct((N_PAD // 4, 128), f32),
    ],
)


def _head_body(agg_ref, tlo_ref, thi_ref, dinv_ref,
               w3lo_ref, w3hi_ref, blo_ref, bhi_ref, s_ref, b3_ref, out_ref):
    dinv = dinv_ref[...]
    x3lo = jax.nn.relu(dinv * (agg_ref[0] + tlo_ref[...]) + blo_ref[...])
    x3hi = jax.nn.relu(dinv * (agg_ref[1] + thi_ref[...]) + bhi_ref[...])
    z = x3lo * w3lo_ref[...] + x3hi * w3hi_ref[...]
    out_ref[...] = (jnp.dot(z, s_ref[...], preferred_element_type=f32)
                    + b3_ref[0, 0])


_head = pl.pallas_call(
    _head_body,
    grid=(GRID,),
    in_specs=[
        pl.BlockSpec((2, RP, 128), lambda i: (0, i, 0)),
        pl.BlockSpec((RP, 128), lambda i: (i, 0)),
        pl.BlockSpec((RP, 128), lambda i: (i, 0)),
        pl.BlockSpec((RP, 128), lambda i: (i, 0)),
        pl.BlockSpec((1, 128), lambda i: (0, 0)),
        pl.BlockSpec((1, 128), lambda i: (0, 0)),
        pl.BlockSpec((1, 128), lambda i: (0, 0)),
        pl.BlockSpec((1, 128), lambda i: (0, 0)),
        pl.BlockSpec((128, 4), lambda i: (0, 0)),
        pl.BlockSpec((1, 8), lambda i: (0, 0)),
    ],
    out_specs=pl.BlockSpec((RP, 4), lambda i: (i, 0)),
    out_shape=jax.ShapeDtypeStruct((N_PAD // 4, 4), f32),
)


def kernel(obs, edge_index, W1, b1, W2, b2, W3, b3):
    src = edge_index[0]
    dst = edge_index[1]
    pad = E_PAD - E
    ar = jnp.arange(pad, dtype=jnp.int32)
    src_p = jnp.concatenate([src, ar % N])
    dst_p = jnp.concatenate([dst, N + ar % (N_PAD - N)])

    obs4 = jnp.pad(obs, ((0, N_PAD - N), (0, 0))).reshape(N_PAD // 4, 4 * D_IN)
    zeros32 = jnp.zeros((N_PAD, 32), f32)
    ones32 = jnp.ones((K, 32), f32)

    eye4 = jnp.eye(4, dtype=f32)
    w1lo = jnp.kron(eye4, W1[:, :32])
    w1hi = jnp.kron(eye4, W1[:, 32:])
    w2aa = jnp.kron(eye4, W2[:32, :32])
    w2ba = jnp.kron(eye4, W2[32:, :32])
    w2ab = jnp.kron(eye4, W2[:32, 32:])
    w2bb = jnp.kron(eye4, W2[32:, 32:])
    b1lo = jnp.tile(b1[:32], 4).reshape(1, 128)
    b1hi = jnp.tile(b1[32:], 4).reshape(1, 128)
    b2lo = jnp.tile(b2[:32], 4).reshape(1, 128)
    b2hi = jnp.tile(b2[32:], 4).reshape(1, 128)
    w3lo = jnp.tile(W3[:32, 0], 4).reshape(1, 128)
    w3hi = jnp.tile(W3[32:, 0], 4).reshape(1, 128)
    ssum = jnp.kron(eye4, jnp.ones((32, 1), f32))
    b3b = jnp.broadcast_to(b3.reshape(1, 1), (1, 8))

    _sc_degree, _sc_scatter = _sc_kernels()
    degp = _sc_degree(dst_p, ones32, zeros32)

    tab1lo, tab1hi, dinv = _scale1(degp.reshape(2, N_PAD // 4, 128),
                                   obs4, w1lo, w1hi)
    agg1 = _sc_scatter(tab1lo.reshape(N_PAD, 32), tab1hi.reshape(N_PAD, 32),
                       src_p, dst_p, zeros32)

    tab2lo, tab2hi = _mid(agg1.reshape(2, N_PAD // 4, 128), tab1lo, tab1hi,
                          dinv, w2aa, w2ba, w2ab, w2bb, b1lo, b1hi)
    agg2 = _sc_scatter(tab2lo.reshape(N_PAD, 32), tab2hi.reshape(N_PAD, 32),
                       src_p, dst_p, zeros32)

    y4 = _head(agg2.reshape(2, N_PAD // 4, 128), tab2lo, tab2hi, dinv,
               w3lo, w3hi, b2lo, b2hi, ssum, b3b)

    y = y4.reshape(-1)[:N]
    return y.reshape(-1, 15)[:, 3:].reshape(-1)


# speedup vs baseline: 1.1179x; 1.0503x over previous
import functools

import jax
import jax.numpy as jnp
from jax import lax
from jax.experimental import pallas as pl
from jax.experimental.pallas import tpu as pltpu
from jax.experimental.pallas import tpu_sc as plsc

N = 49995
E = 799920
D_IN = 128
D_H = 64

N_PAD = 50176
ROWS_PER_TILE = N_PAD // 16

K = 288
CH = 174
E_PAD = 16 * K * CH
DCH = E_PAD // (32 * K)

f32 = jnp.float32


def _sc_degree_body(dst_hbm, ones_hbm, zeros_hbm, out_hbm,
                    idx_v0, idx_v1, idx_v2, ones_v, deg_sh,
                    is0, is1, is2, ss0, ss1, ss2):
    c = lax.axis_index("c")
    s = lax.axis_index("s")
    pltpu.sync_copy(ones_hbm, ones_v)
    r0 = s * ROWS_PER_TILE
    pltpu.sync_copy(zeros_hbm.at[pl.ds(r0, ROWS_PER_TILE)],
                    deg_sh.at[pl.ds(r0, ROWS_PER_TILE)])
    plsc.subcore_barrier()
    base = (c * 16 + s) * DCH * K
    bufs = ((idx_v0, is0, ss0), (idx_v1, is1, ss1), (idx_v2, is2, ss2))

    def stage_idx(i, t):
        iv, isem, _ = bufs[t]
        pltpu.async_copy(dst_hbm.at[pl.ds(base + i * K, K)], iv, isem)

    def scatter(i, t):
        iv, isem, ss = bufs[t]
        pltpu.make_async_copy(dst_hbm.at[pl.ds(base + i * K, K)], iv,
                              isem).wait()
        pltpu.async_copy(ones_v, deg_sh.at[iv], ss, add=True)

    def wait_scatter(t):
        iv, _, ss = bufs[t]
        pltpu.make_async_copy(ones_v, deg_sh.at[iv], ss).wait()

    stage_idx(0, 0)
    stage_idx(1, 1)
    stage_idx(2, 2)
    scatter(0, 0)
    scatter(1, 1)

    peel = 3 + (DCH - 2) % 3
    loop_n = (DCH - 2 - peel) // 3

    @pl.loop(0, loop_n)
    def _outer(o):
        for u in range(3):
            i = 2 + 3 * o + u
            t = (2 + u) % 3
            wait_scatter(u % 3)
            stage_idx(i + 1, u % 3)
            scatter(i, t)

    for u in range(peel):
        i = 2 + 3 * loop_n + u
        t = (2 + u) % 3
        wait_scatter(u % 3)
        if i + 1 < DCH:
            stage_idx(i + 1, u % 3)
        scatter(i, t)

    wait_scatter((DCH - 2) % 3)
    wait_scatter((DCH - 1) % 3)
    plsc.subcore_barrier()
    pltpu.sync_copy(deg_sh.at[pl.ds(r0, ROWS_PER_TILE)],
                    out_hbm.at[c, pl.ds(r0, ROWS_PER_TILE)])


def _make_sc_scatter_body():
    def body(tlo_hbm, thi_hbm, src_hbm, dst_hbm, zeros_hbm, out_hbm,
             src_v0, dst_v0, src_v1, dst_v1, src_v2, dst_v2,
             rows_v0, rows_v1, rows_v2, agg_sh,
             is0, is1, is2, gs0, gs1, gs2, ss0, ss1, ss2):
        c = lax.axis_index("c")
        s = lax.axis_index("s")
        r0 = s * ROWS_PER_TILE
        pltpu.sync_copy(zeros_hbm.at[pl.ds(r0, ROWS_PER_TILE)],
                        agg_sh.at[pl.ds(r0, ROWS_PER_TILE)])
        plsc.subcore_barrier()
        base = s * CH * K
        bufs = ((src_v0, dst_v0, rows_v0, is0, gs0, ss0),
                (src_v1, dst_v1, rows_v1, is1, gs1, ss1),
                (src_v2, dst_v2, rows_v2, is2, gs2, ss2))

        def stage_idx(i, t):
            sv, dv, rv, isem, gsem, ssem = bufs[t]
            pltpu.async_copy(src_hbm.at[pl.ds(base + i * K, K)], sv, isem)
            pltpu.async_copy(dst_hbm.at[pl.ds(base + i * K, K)], dv, isem)

        def wait_idx(i, t):
            sv, dv, rv, isem, gsem, ssem = bufs[t]
            pltpu.make_async_copy(src_hbm.at[pl.ds(base + i * K, K)], sv,
                                  isem).wait()
            pltpu.make_async_copy(dst_hbm.at[pl.ds(base + i * K, K)], dv,
                                  isem).wait()

        def run(tab):
            def start_gather(i, t):
                sv, dv, rv, isem, gsem, ssem = bufs[t]
                wait_idx(i, t)
                pltpu.async_copy(tab.at[sv], rv, gsem)

            def finish_scatter(t):
                sv, dv, rv, isem, gsem, ssem = bufs[t]
                pltpu.make_async_copy(tab.at[sv], rv, gsem).wait()
                pltpu.async_copy(rv, agg_sh.at[dv], ssem, add=True)

            def wait_scatter(t):
                sv, dv, rv, isem, gsem, ssem = bufs[t]
                pltpu.make_async_copy(rv, agg_sh.at[dv], ssem).wait()

            stage_idx(0, 0)
            stage_idx(1, 1)
            stage_idx(2, 2)
            start_gather(0, 0)
            start_gather(1, 1)
            finish_scatter(0)
            wait_scatter(0)
            stage_idx(3, 0)
            start_gather(2, 2)
            finish_scatter(1)

            @pl.loop(0, (CH - 6) // 3)
            def _outer(o):
                for u in range(3):
                    i = 3 + 3 * o + u
                    t = u
                    wait_scatter((u + 1) % 3)
                    stage_idx(i + 1, (u + 1) % 3)
                    start_gather(i, t)
                    finish_scatter((u + 2) % 3)

            for i in range(CH - 3, CH):
                t = i % 3
                wait_scatter((i + 1) % 3)
                if i + 1 < CH:
                    stage_idx(i + 1, (i + 1) % 3)
                start_gather(i, t)
                finish_scatter((i + 2) % 3)

            finish_scatter((CH - 1) % 3)
            wait_scatter((CH - 2) % 3)
            wait_scatter((CH - 1) % 3)

        @pl.when(c == 0)
        def _lo():
            run(tlo_hbm)

        @pl.when(c == 1)
        def _hi():
            run(thi_hbm)

        plsc.subcore_barrier()
        pltpu.sync_copy(agg_sh.at[pl.ds(r0, ROWS_PER_TILE)],
                        out_hbm.at[c, pl.ds(r0, ROWS_PER_TILE)])

    return body


@functools.cache
def _sc_kernels():
    mesh = plsc.VectorSubcoreMesh(core_axis_name="c", subcore_axis_name="s")
    params = pltpu.CompilerParams(use_tc_tiling_on_sc=False)
    sc_degree = pl.kernel(
        _sc_degree_body,
        out_type=jax.ShapeDtypeStruct((2, N_PAD, 32), f32),
        mesh=mesh,
        compiler_params=params,
        scratch_types=(
            [pltpu.VMEM((K,), jnp.int32)] * 3
            + [pltpu.VMEM((K, 32), f32)]
            + [pltpu.VMEM_SHARED((N_PAD, 32), f32)]
            + [pltpu.SemaphoreType.DMA] * 6
        ),
    )
    sc_scatter = pl.kernel(
        _make_sc_scatter_body(),
        out_type=jax.ShapeDtypeStruct((2, N_PAD, 32), f32),
        mesh=mesh,
        compiler_params=params,
        scratch_types=(
            [pltpu.VMEM((K,), jnp.int32)] * 6
            + [pltpu.VMEM((K, 32), f32)] * 3
            + [pltpu.VMEM_SHARED((N_PAD, 32), f32)]
            + [pltpu.SemaphoreType.DMA] * 9
        ),
    )
    return sc_degree, sc_scatter


RB = 512
RP = RB // 4
GRID = N_PAD // RB


def _scale1_body(d_ref, x_ref, wlo_ref, whi_ref, tlo_ref, thi_ref, dinv_ref):
    dinv = lax.rsqrt(d_ref[0] + d_ref[1] + 1.0)
    x4 = x_ref[...]
    tlo_ref[...] = dinv * jnp.dot(x4, wlo_ref[...], preferred_element_type=f32)
    thi_ref[...] = dinv * jnp.dot(x4, whi_ref[...], preferred_element_type=f32)
    dinv_ref[...] = dinv


_scale1 = pl.pallas_call(
    _scale1_body,
    grid=(GRID,),
    in_specs=[
        pl.BlockSpec((2, RP, 128), lambda i: (0, i, 0)),
        pl.BlockSpec((RP, 4 * D_IN), lambda i: (i, 0)),
        pl.BlockSpec((4 * D_IN, 128), lambda i: (0, 0)),
        pl.BlockSpec((4 * D_IN, 128), lambda i: (0, 0)),
    ],
    out_specs=[
        pl.BlockSpec((RP, 128), lambda i: (i, 0)),
        pl.BlockSpec((RP, 128), lambda i: (i, 0)),
        pl.BlockSpec((RP, 128), lambda i: (i, 0)),
    ],
    out_shape=[
        jax.ShapeDtypeStruct((N_PAD // 4, 128), f32),
        jax.ShapeDtypeStruct((N_PAD // 4, 128), f32),
        jax.ShapeDtypeStruct((N_PAD // 4, 128), f32),
    ],
)


def _mid_body(agg_ref, tlo_ref, thi_ref, dinv_ref,
              waa_ref, wba_ref, wab_ref, wbb_ref, blo_ref, bhi_ref,
              olo_ref, ohi_ref):
    dinv = dinv_ref[...]
    x2lo = jax.nn.relu(dinv * (agg_ref[0] + tlo_ref[...]) + blo_ref[...])
    x2hi = jax.nn.relu(dinv * (agg_ref[1] + thi_ref[...]) + bhi_ref[...])
    h2lo = (jnp.dot(x2lo, waa_ref[...], preferred_element_type=f32)
            + jnp.dot(x2hi, wba_ref[...], preferred_element_type=f32))
    h2hi = (jnp.dot(x2lo, wab_ref[...], preferred_element_type=f32)
            + jnp.dot(x2hi, wbb_ref[...], preferred_element_type=f32))
    olo_ref[...] = dinv * h2lo
    ohi_ref[...] = dinv * h2hi


_mid = pl.pallas_call(
    _mid_body,
    grid=(GRID,),
    in_specs=[
        pl.BlockSpec((2, RP, 128), lambda i: (0, i, 0)),
        pl.BlockSpec((RP, 128), lambda i: (i, 0)),
        pl.BlockSpec((RP, 128), lambda i: (i, 0)),
        pl.BlockSpec((RP, 128), lambda i: (i, 0)),
        pl.BlockSpec((128, 128), lambda i: (0, 0)),
        pl.BlockSpec((128, 128), lambda i: (0, 0)),
        pl.BlockSpec((128, 128), lambda i: (0, 0)),
        pl.BlockSpec((128, 128), lambda i: (0, 0)),
        pl.BlockSpec((1, 128), lambda i: (0, 0)),
        pl.BlockSpec((1, 128), lambda i: (0, 0)),
    ],
    out_specs=[
        pl.BlockSpec((RP, 128), lambda i: (i, 0)),
        pl.BlockSpec((RP, 128), lambda i: (i, 0)),
    ],
    out_shape=[
        jax.ShapeDtypeStruct((N_PAD // 4, 128), f32),
        jax.ShapeDtypeStruct((N_PAD // 4, 128), f32),
    ],
)


def _head_body(agg_ref, tlo_ref, thi_ref, dinv_ref,
               w3lo_ref, w3hi_ref, blo_ref, bhi_ref, s_ref, b3_ref, out_ref):
    dinv = dinv_ref[...]
    x3lo = jax.nn.relu(dinv * (agg_ref[0] + tlo_ref[...]) + blo_ref[...])
    x3hi = jax.nn.relu(dinv * (agg_ref[1] + thi_ref[...]) + bhi_ref[...])
    z = x3lo * w3lo_ref[...] + x3hi * w3hi_ref[...]
    out_ref[...] = (jnp.dot(z, s_ref[...], preferred_element_type=f32)
                    + b3_ref[0, 0])


_head = pl.pallas_call(
    _head_body,
    grid=(GRID,),
    in_specs=[
        pl.BlockSpec((2, RP, 128), lambda i: (0, i, 0)),
        pl.BlockSpec((RP, 128), lambda i: (i, 0)),
        pl.BlockSpec((RP, 128), lambda i: (i, 0)),
        pl.BlockSpec((RP, 128), lambda i: (i, 0)),
        pl.BlockSpec((1, 128), lambda i: (0, 0)),
        pl.BlockSpec((1, 128), lambda i: (0, 0)),
        pl.BlockSpec((1, 128), lambda i: (0, 0)),
        pl.BlockSpec((1, 128), lambda i: (0, 0)),
        pl.BlockSpec((128, 4), lambda i: (0, 0)),
        pl.BlockSpec((1, 8), lambda i: (0, 0)),
    ],
    out_specs=pl.BlockSpec((RP, 4), lambda i: (i, 0)),
    out_shape=jax.ShapeDtypeStruct((N_PAD // 4, 4), f32),
)


def kernel(obs, edge_index, W1, b1, W2, b2, W3, b3):
    src = edge_index[0]
    dst = edge_index[1]
    pad = E_PAD - E
    ar = jnp.arange(pad, dtype=jnp.int32)
    src_p = jnp.concatenate([src, ar % N])
    dst_p = jnp.concatenate([dst, N + ar % (N_PAD - N)])

    obs4 = jnp.pad(obs, ((0, N_PAD - N), (0, 0))).reshape(N_PAD // 4, 4 * D_IN)
    zeros32 = jnp.zeros((N_PAD, 32), f32)
    ones32 = jnp.ones((K, 32), f32)

    eye4 = jnp.eye(4, dtype=f32)
    w1lo = jnp.kron(eye4, W1[:, :32])
    w1hi = jnp.kron(eye4, W1[:, 32:])
    w2aa = jnp.kron(eye4, W2[:32, :32])
    w2ba = jnp.kron(eye4, W2[32:, :32])
    w2ab = jnp.kron(eye4, W2[:32, 32:])
    w2bb = jnp.kron(eye4, W2[32:, 32:])
    b1lo = jnp.tile(b1[:32], 4).reshape(1, 128)
    b1hi = jnp.tile(b1[32:], 4).reshape(1, 128)
    b2lo = jnp.tile(b2[:32], 4).reshape(1, 128)
    b2hi = jnp.tile(b2[32:], 4).reshape(1, 128)
    w3lo = jnp.tile(W3[:32, 0], 4).reshape(1, 128)
    w3hi = jnp.tile(W3[32:, 0], 4).reshape(1, 128)
    ssum = jnp.kron(eye4, jnp.ones((32, 1), f32))
    b3b = jnp.broadcast_to(b3.reshape(1, 1), (1, 8))

    _sc_degree, _sc_scatter = _sc_kernels()
    degp = _sc_degree(dst_p, ones32, zeros32)

    tab1lo, tab1hi, dinv = _scale1(degp.reshape(2, N_PAD // 4, 128),
                                   obs4, w1lo, w1hi)
    agg1 = _sc_scatter(tab1lo.reshape(N_PAD, 32), tab1hi.reshape(N_PAD, 32),
                       src_p, dst_p, zeros32)

    tab2lo, tab2hi = _mid(agg1.reshape(2, N_PAD // 4, 128), tab1lo, tab1hi,
                          dinv, w2aa, w2ba, w2ab, w2bb, b1lo, b1hi)
    agg2 = _sc_scatter(tab2lo.reshape(N_PAD, 32), tab2hi.reshape(N_PAD, 32),
                       src_p, dst_p, zeros32)

    y4 = _head(agg2.reshape(2, N_PAD // 4, 128), tab2lo, tab2hi, dinv,
               w3lo, w3hi, b2lo, b2hi, ssum, b3b)

    y = y4.reshape(-1)[:N]
    return y.reshape(-1, 15)[:, 3:].reshape(-1)


# speedup vs baseline: 1.1238x; 1.0053x over previous
import functools

import jax
import jax.numpy as jnp
from jax import lax
from jax.experimental import pallas as pl
from jax.experimental.pallas import tpu as pltpu
from jax.experimental.pallas import tpu_sc as plsc

N = 49995
E = 799920
D_IN = 128
D_H = 64

N_PAD = 50176
ROWS_PER_TILE = N_PAD // 16

K = 288
CH = 174
E_PAD = 16 * K * CH
DCH = E_PAD // (32 * K)

f32 = jnp.float32


def _sc_degree_body(dst_hbm, ones_hbm, zeros_hbm, out_hbm,
                    idx_v0, idx_v1, idx_v2, ones_v, deg_sh,
                    is0, is1, is2, ss0, ss1, ss2):
    c = lax.axis_index("c")
    s = lax.axis_index("s")
    pltpu.sync_copy(ones_hbm, ones_v)
    r0 = s * ROWS_PER_TILE
    pltpu.sync_copy(zeros_hbm.at[pl.ds(r0, ROWS_PER_TILE)],
                    deg_sh.at[pl.ds(r0, ROWS_PER_TILE)])
    plsc.subcore_barrier()
    base = (c * 16 + s) * DCH * K
    bufs = ((idx_v0, is0, ss0), (idx_v1, is1, ss1), (idx_v2, is2, ss2))

    def stage_idx(i, t):
        iv, isem, _ = bufs[t]
        pltpu.async_copy(dst_hbm.at[pl.ds(base + i * K, K)], iv, isem)

    def scatter(i, t):
        iv, isem, ss = bufs[t]
        pltpu.make_async_copy(dst_hbm.at[pl.ds(base + i * K, K)], iv,
                              isem).wait()
        pltpu.async_copy(ones_v, deg_sh.at[iv], ss, add=True)

    def wait_scatter(t):
        iv, _, ss = bufs[t]
        pltpu.make_async_copy(ones_v, deg_sh.at[iv], ss).wait()

    stage_idx(0, 0)
    stage_idx(1, 1)
    stage_idx(2, 2)
    scatter(0, 0)
    scatter(1, 1)

    peel = 3 + (DCH - 2) % 3
    loop_n = (DCH - 2 - peel) // 3

    @pl.loop(0, loop_n)
    def _outer(o):
        for u in range(3):
            i = 2 + 3 * o + u
            t = (2 + u) % 3
            wait_scatter(u % 3)
            stage_idx(i + 1, u % 3)
            scatter(i, t)

    for u in range(peel):
        i = 2 + 3 * loop_n + u
        t = (2 + u) % 3
        wait_scatter(u % 3)
        if i + 1 < DCH:
            stage_idx(i + 1, u % 3)
        scatter(i, t)

    wait_scatter((DCH - 2) % 3)
    wait_scatter((DCH - 1) % 3)
    plsc.subcore_barrier()
    pltpu.sync_copy(deg_sh.at[pl.ds(r0, ROWS_PER_TILE)],
                    out_hbm.at[c, pl.ds(r0, ROWS_PER_TILE)])


def _make_sc_scatter_body():
    def body(tlo_hbm, thi_hbm, src_hbm, dst_hbm, zeros_hbm, out_hbm,
             src_v0, dst_v0, src_v1, dst_v1, src_v2, dst_v2,
             rows_v0, rows_v1, rows_v2, agg_sh,
             is0, is1, is2, gs0, gs1, gs2, ss0, ss1, ss2):
        c = lax.axis_index("c")
        s = lax.axis_index("s")
        r0 = s * ROWS_PER_TILE
        pltpu.sync_copy(zeros_hbm.at[pl.ds(r0, ROWS_PER_TILE)],
                        agg_sh.at[pl.ds(r0, ROWS_PER_TILE)])
        plsc.subcore_barrier()
        base = s * CH * K
        bufs = ((src_v0, dst_v0, rows_v0, is0, gs0, ss0),
                (src_v1, dst_v1, rows_v1, is1, gs1, ss1),
                (src_v2, dst_v2, rows_v2, is2, gs2, ss2))

        def stage_idx(i, t):
            sv, dv, rv, isem, gsem, ssem = bufs[t]
            pltpu.async_copy(src_hbm.at[pl.ds(base + i * K, K)], sv, isem)
            pltpu.async_copy(dst_hbm.at[pl.ds(base + i * K, K)], dv, isem)

        def wait_idx(i, t):
            sv, dv, rv, isem, gsem, ssem = bufs[t]
            pltpu.make_async_copy(src_hbm.at[pl.ds(base + i * K, K)], sv,
                                  isem).wait()
            pltpu.make_async_copy(dst_hbm.at[pl.ds(base + i * K, K)], dv,
                                  isem).wait()

        def run(tab):
            def start_gather(i, t):
                sv, dv, rv, isem, gsem, ssem = bufs[t]
                wait_idx(i, t)
                pltpu.async_copy(tab.at[sv], rv, gsem)

            def finish_scatter(t):
                sv, dv, rv, isem, gsem, ssem = bufs[t]
                pltpu.make_async_copy(tab.at[sv], rv, gsem).wait()
                pltpu.async_copy(rv, agg_sh.at[dv], ssem, add=True)

            def wait_scatter(t):
                sv, dv, rv, isem, gsem, ssem = bufs[t]
                pltpu.make_async_copy(rv, agg_sh.at[dv], ssem).wait()

            stage_idx(0, 0)
            stage_idx(1, 1)
            stage_idx(2, 2)
            start_gather(0, 0)
            start_gather(1, 1)
            finish_scatter(0)
            wait_scatter(0)
            stage_idx(3, 0)
            start_gather(2, 2)
            finish_scatter(1)

            @pl.loop(0, (CH - 6) // 3)
            def _outer(o):
                for u in range(3):
                    i = 3 + 3 * o + u
                    t = u
                    wait_scatter((u + 1) % 3)
                    stage_idx(i + 1, (u + 1) % 3)
                    start_gather(i, t)
                    finish_scatter((u + 2) % 3)

            for i in range(CH - 3, CH):
                t = i % 3
                wait_scatter((i + 1) % 3)
                if i + 1 < CH:
                    stage_idx(i + 1, (i + 1) % 3)
                start_gather(i, t)
                finish_scatter((i + 2) % 3)

            finish_scatter((CH - 1) % 3)
            wait_scatter((CH - 2) % 3)
            wait_scatter((CH - 1) % 3)

        @pl.when(c == 0)
        def _lo():
            run(tlo_hbm)

        @pl.when(c == 1)
        def _hi():
            run(thi_hbm)

        plsc.subcore_barrier()
        pltpu.sync_copy(agg_sh.at[pl.ds(r0, ROWS_PER_TILE)],
                        out_hbm.at[c, pl.ds(r0, ROWS_PER_TILE)])

    return body


@functools.cache
def _sc_kernels():
    mesh = plsc.VectorSubcoreMesh(core_axis_name="c", subcore_axis_name="s")
    params = pltpu.CompilerParams(use_tc_tiling_on_sc=False)
    sc_degree = pl.kernel(
        _sc_degree_body,
        out_type=jax.ShapeDtypeStruct((2, N_PAD, 32), f32),
        mesh=mesh,
        compiler_params=params,
        scratch_types=(
            [pltpu.VMEM((K,), jnp.int32)] * 3
            + [pltpu.VMEM((K, 32), f32)]
            + [pltpu.VMEM_SHARED((N_PAD, 32), f32)]
            + [pltpu.SemaphoreType.DMA] * 6
        ),
    )
    sc_scatter = pl.kernel(
        _make_sc_scatter_body(),
        out_type=jax.ShapeDtypeStruct((2, N_PAD, 32), f32),
        mesh=mesh,
        compiler_params=params,
        scratch_types=(
            [pltpu.VMEM((K,), jnp.int32)] * 6
            + [pltpu.VMEM((K, 32), f32)] * 3
            + [pltpu.VMEM_SHARED((N_PAD, 32), f32)]
            + [pltpu.SemaphoreType.DMA] * 9
        ),
    )
    return sc_degree, sc_scatter


RB = 512
RP = RB // 4
GRID = N_PAD // RB


def _scale1_body(d_ref, x_ref, wlo_ref, whi_ref, tlo_ref, thi_ref, dinv_ref):
    dinv = lax.rsqrt(d_ref[0] + d_ref[1] + 1.0)
    x4 = x_ref[...]
    tlo_ref[...] = dinv * jnp.dot(x4, wlo_ref[...], preferred_element_type=f32)
    thi_ref[...] = dinv * jnp.dot(x4, whi_ref[...], preferred_element_type=f32)
    dinv_ref[...] = dinv


_scale1 = pl.pallas_call(
    _scale1_body,
    grid=(GRID,),
    in_specs=[
        pl.BlockSpec((2, RP, 128), lambda i: (0, i, 0)),
        pl.BlockSpec((RP, 4 * D_IN), lambda i: (i, 0)),
        pl.BlockSpec((4 * D_IN, 128), lambda i: (0, 0)),
        pl.BlockSpec((4 * D_IN, 128), lambda i: (0, 0)),
    ],
    out_specs=[
        pl.BlockSpec((RP, 128), lambda i: (i, 0)),
        pl.BlockSpec((RP, 128), lambda i: (i, 0)),
        pl.BlockSpec((RP, 128), lambda i: (i, 0)),
    ],
    out_shape=[
        jax.ShapeDtypeStruct((N_PAD // 4, 128), f32),
        jax.ShapeDtypeStruct((N_PAD // 4, 128), f32),
        jax.ShapeDtypeStruct((N_PAD // 4, 128), f32),
    ],
)


def _mid_body(agg_ref, tlo_ref, thi_ref, dinv_ref,
              waa_ref, wba_ref, wab_ref, wbb_ref, blo_ref, bhi_ref,
              olo_ref, ohi_ref):
    dinv = dinv_ref[...]
    x2lo = jax.nn.relu(dinv * (agg_ref[0] + tlo_ref[...]) + blo_ref[...])
    x2hi = jax.nn.relu(dinv * (agg_ref[1] + thi_ref[...]) + bhi_ref[...])
    h2lo = (jnp.dot(x2lo, waa_ref[...], preferred_element_type=f32)
            + jnp.dot(x2hi, wba_ref[...], preferred_element_type=f32))
    h2hi = (jnp.dot(x2lo, wab_ref[...], preferred_element_type=f32)
            + jnp.dot(x2hi, wbb_ref[...], preferred_element_type=f32))
    olo_ref[...] = dinv * h2lo
    ohi_ref[...] = dinv * h2hi


_mid = pl.pallas_call(
    _mid_body,
    grid=(GRID,),
    in_specs=[
        pl.BlockSpec((2, RP, 128), lambda i: (0, i, 0)),
        pl.BlockSpec((RP, 128), lambda i: (i, 0)),
        pl.BlockSpec((RP, 128), lambda i: (i, 0)),
        pl.BlockSpec((RP, 128), lambda i: (i, 0)),
        pl.BlockSpec((128, 128), lambda i: (0, 0)),
        pl.BlockSpec((128, 128), lambda i: (0, 0)),
        pl.BlockSpec((128, 128), lambda i: (0, 0)),
        pl.BlockSpec((128, 128), lambda i: (0, 0)),
        pl.BlockSpec((1, 128), lambda i: (0, 0)),
        pl.BlockSpec((1, 128), lambda i: (0, 0)),
    ],
    out_specs=[
        pl.BlockSpec((RP, 128), lambda i: (i, 0)),
        pl.BlockSpec((RP, 128), lambda i: (i, 0)),
    ],
    out_shape=[
        jax.ShapeDtypeStruct((N_PAD // 4, 128), f32),
        jax.ShapeDtypeStruct((N_PAD // 4, 128), f32),
    ],
)


def _head_body(agg_ref, tlo_ref, thi_ref, dinv_ref,
               w3lo_ref, w3hi_ref, blo_ref, bhi_ref, s_ref, b3_ref, out_ref):
    dinv = dinv_ref[...]
    x3lo = jax.nn.relu(dinv * (agg_ref[0] + tlo_ref[...]) + blo_ref[...])
    x3hi = jax.nn.relu(dinv * (agg_ref[1] + thi_ref[...]) + bhi_ref[...])
    z = x3lo * w3lo_ref[...] + x3hi * w3hi_ref[...]
    out_ref[...] = (jnp.dot(z, s_ref[...], preferred_element_type=f32)
                    + b3_ref[0, 0])


_head = pl.pallas_call(
    _head_body,
    grid=(GRID,),
    in_specs=[
        pl.BlockSpec((2, RP, 128), lambda i: (0, i, 0)),
        pl.BlockSpec((RP, 128), lambda i: (i, 0)),
        pl.BlockSpec((RP, 128), lambda i: (i, 0)),
        pl.BlockSpec((RP, 128), lambda i: (i, 0)),
        pl.BlockSpec((1, 128), lambda i: (0, 0)),
        pl.BlockSpec((1, 128), lambda i: (0, 0)),
        pl.BlockSpec((1, 128), lambda i: (0, 0)),
        pl.BlockSpec((1, 128), lambda i: (0, 0)),
        pl.BlockSpec((128, 4), lambda i: (0, 0)),
        pl.BlockSpec((1, 8), lambda i: (0, 0)),
    ],
    out_specs=pl.BlockSpec((RP, 4), lambda i: (i, 0)),
    out_shape=jax.ShapeDtypeStruct((N_PAD // 4, 4), f32),
)


ECB = 27648


def _edges_body(ei_ref, src_ref, dst_ref):
    i = pl.program_id(0)
    col = i * ECB + lax.broadcasted_iota(jnp.int32, (ECB,), 0)
    real = col < E
    src_ref[...] = jnp.where(real, ei_ref[0], col % N)
    dst_ref[...] = jnp.where(real, ei_ref[1], N + col % (N_PAD - N))


_edges = pl.pallas_call(
    _edges_body,
    grid=(E_PAD // ECB,),
    in_specs=[pl.BlockSpec((2, ECB), lambda i: (0, i))],
    out_specs=[
        pl.BlockSpec((ECB,), lambda i: (i,)),
        pl.BlockSpec((ECB,), lambda i: (i,)),
    ],
    out_shape=[
        jax.ShapeDtypeStruct((E_PAD,), jnp.int32),
        jax.ShapeDtypeStruct((E_PAD,), jnp.int32),
    ],
)


def kernel(obs, edge_index, W1, b1, W2, b2, W3, b3):
    src_p, dst_p = _edges(edge_index)

    obs4 = jnp.pad(obs, ((0, N_PAD - N), (0, 0))).reshape(N_PAD // 4, 4 * D_IN)
    zeros32 = jnp.zeros((N_PAD, 32), f32)
    ones32 = jnp.ones((K, 32), f32)

    eye4 = jnp.eye(4, dtype=f32)
    w1lo = jnp.kron(eye4, W1[:, :32])
    w1hi = jnp.kron(eye4, W1[:, 32:])
    w2aa = jnp.kron(eye4, W2[:32, :32])
    w2ba = jnp.kron(eye4, W2[32:, :32])
    w2ab = jnp.kron(eye4, W2[:32, 32:])
    w2bb = jnp.kron(eye4, W2[32:, 32:])
    b1lo = jnp.tile(b1[:32], 4).reshape(1, 128)
    b1hi = jnp.tile(b1[32:], 4).reshape(1, 128)
    b2lo = jnp.tile(b2[:32], 4).reshape(1, 128)
    b2hi = jnp.tile(b2[32:], 4).reshape(1, 128)
    w3lo = jnp.tile(W3[:32, 0], 4).reshape(1, 128)
    w3hi = jnp.tile(W3[32:, 0], 4).reshape(1, 128)
    ssum = jnp.kron(eye4, jnp.ones((32, 1), f32))
    b3b = jnp.broadcast_to(b3.reshape(1, 1), (1, 8))

    _sc_degree, _sc_scatter = _sc_kernels()
    degp = _sc_degree(dst_p, ones32, zeros32)

    tab1lo, tab1hi, dinv = _scale1(degp.reshape(2, N_PAD // 4, 128),
                                   obs4, w1lo, w1hi)
    agg1 = _sc_scatter(tab1lo.reshape(N_PAD, 32), tab1hi.reshape(N_PAD, 32),
                       src_p, dst_p, zeros32)

    tab2lo, tab2hi = _mid(agg1.reshape(2, N_PAD // 4, 128), tab1lo, tab1hi,
                          dinv, w2aa, w2ba, w2ab, w2bb, b1lo, b1hi)
    agg2 = _sc_scatter(tab2lo.reshape(N_PAD, 32), tab2hi.reshape(N_PAD, 32),
                       src_p, dst_p, zeros32)

    y4 = _head(agg2.reshape(2, N_PAD // 4, 128), tab2lo, tab2hi, dinv,
               w3lo, w3hi, b2lo, b2hi, ssum, b3b)

    y = y4.reshape(-1)[:N]
    return y.reshape(-1, 15)[:, 3:].reshape(-1)


# speedup vs baseline: 1.2746x; 1.1341x over previous
import functools

import jax
import jax.numpy as jnp
from jax import lax
from jax.experimental import pallas as pl
from jax.experimental.pallas import tpu as pltpu
from jax.experimental.pallas import tpu_sc as plsc

N = 49995
E = 799920
D_IN = 128
D_H = 64

N_PAD = 50176
ROWS_PER_TILE = N_PAD // 16

K = 288
CH = 174
E_PAD = 16 * K * CH
DCH = E_PAD // (32 * K)

f32 = jnp.float32


def _sc_degree_body(dst_hbm, ones_hbm, zeros_hbm, out_hbm,
                    idx_v0, idx_v1, idx_v2, ones_v, deg_sh,
                    is0, is1, is2, ss0, ss1, ss2):
    c = lax.axis_index("c")
    s = lax.axis_index("s")
    pltpu.sync_copy(ones_hbm, ones_v)
    r0 = s * ROWS_PER_TILE
    pltpu.sync_copy(zeros_hbm.at[pl.ds(r0, ROWS_PER_TILE)],
                    deg_sh.at[pl.ds(r0, ROWS_PER_TILE)])
    plsc.subcore_barrier()
    base = (c * 16 + s) * DCH * K
    bufs = ((idx_v0, is0, ss0), (idx_v1, is1, ss1), (idx_v2, is2, ss2))

    def stage_idx(i, t):
        iv, isem, _ = bufs[t]
        pltpu.async_copy(dst_hbm.at[pl.ds(base + i * K, K)], iv, isem)

    def scatter(i, t):
        iv, isem, ss = bufs[t]
        pltpu.make_async_copy(dst_hbm.at[pl.ds(base + i * K, K)], iv,
                              isem).wait()
        pltpu.async_copy(ones_v, deg_sh.at[iv], ss, add=True)

    def wait_scatter(t):
        iv, _, ss = bufs[t]
        pltpu.make_async_copy(ones_v, deg_sh.at[iv], ss).wait()

    stage_idx(0, 0)
    stage_idx(1, 1)
    stage_idx(2, 2)
    scatter(0, 0)
    scatter(1, 1)

    peel = 3 + (DCH - 2) % 3
    loop_n = (DCH - 2 - peel) // 3

    @pl.loop(0, loop_n)
    def _outer(o):
        for u in range(3):
            i = 2 + 3 * o + u
            t = (2 + u) % 3
            wait_scatter(u % 3)
            stage_idx(i + 1, u % 3)
            scatter(i, t)

    for u in range(peel):
        i = 2 + 3 * loop_n + u
        t = (2 + u) % 3
        wait_scatter(u % 3)
        if i + 1 < DCH:
            stage_idx(i + 1, u % 3)
        scatter(i, t)

    wait_scatter((DCH - 2) % 3)
    wait_scatter((DCH - 1) % 3)
    plsc.subcore_barrier()
    pltpu.sync_copy(deg_sh.at[pl.ds(r0, ROWS_PER_TILE)],
                    out_hbm.at[c, pl.ds(r0, ROWS_PER_TILE)])


def _make_sc_scatter_body():
    def body(tlo_hbm, thi_hbm, src_hbm, dst_hbm, zeros_hbm, out_hbm,
             src_v0, dst_v0, src_v1, dst_v1, src_v2, dst_v2,
             rows_v0, rows_v1, rows_v2, agg_sh,
             is0, is1, is2, gs0, gs1, gs2, ss0, ss1, ss2):
        c = lax.axis_index("c")
        s = lax.axis_index("s")
        r0 = s * ROWS_PER_TILE
        pltpu.sync_copy(zeros_hbm.at[pl.ds(r0, ROWS_PER_TILE)],
                        agg_sh.at[pl.ds(r0, ROWS_PER_TILE)])
        plsc.subcore_barrier()
        base = s * CH * K
        bufs = ((src_v0, dst_v0, rows_v0, is0, gs0, ss0),
                (src_v1, dst_v1, rows_v1, is1, gs1, ss1),
                (src_v2, dst_v2, rows_v2, is2, gs2, ss2))

        def stage_idx(i, t):
            sv, dv, rv, isem, gsem, ssem = bufs[t]
            pltpu.async_copy(src_hbm.at[pl.ds(base + i * K, K)], sv, isem)
            pltpu.async_copy(dst_hbm.at[pl.ds(base + i * K, K)], dv, isem)

        def wait_idx(i, t):
            sv, dv, rv, isem, gsem, ssem = bufs[t]
            pltpu.make_async_copy(src_hbm.at[pl.ds(base + i * K, K)], sv,
                                  isem).wait()
            pltpu.make_async_copy(dst_hbm.at[pl.ds(base + i * K, K)], dv,
                                  isem).wait()

        def run(tab):
            def start_gather(i, t):
                sv, dv, rv, isem, gsem, ssem = bufs[t]
                wait_idx(i, t)
                pltpu.async_copy(tab.at[sv], rv, gsem)

            def finish_scatter(t):
                sv, dv, rv, isem, gsem, ssem = bufs[t]
                pltpu.make_async_copy(tab.at[sv], rv, gsem).wait()
                pltpu.async_copy(rv, agg_sh.at[dv], ssem, add=True)

            def wait_scatter(t):
                sv, dv, rv, isem, gsem, ssem = bufs[t]
                pltpu.make_async_copy(rv, agg_sh.at[dv], ssem).wait()

            stage_idx(0, 0)
            stage_idx(1, 1)
            stage_idx(2, 2)
            start_gather(0, 0)
            start_gather(1, 1)
            finish_scatter(0)
            wait_scatter(0)
            stage_idx(3, 0)
            start_gather(2, 2)
            finish_scatter(1)

            @pl.loop(0, (CH - 6) // 3)
            def _outer(o):
                for u in range(3):
                    i = 3 + 3 * o + u
                    t = u
                    wait_scatter((u + 1) % 3)
                    stage_idx(i + 1, (u + 1) % 3)
                    start_gather(i, t)
                    finish_scatter((u + 2) % 3)

            for i in range(CH - 3, CH):
                t = i % 3
                wait_scatter((i + 1) % 3)
                if i + 1 < CH:
                    stage_idx(i + 1, (i + 1) % 3)
                start_gather(i, t)
                finish_scatter((i + 2) % 3)

            finish_scatter((CH - 1) % 3)
            wait_scatter((CH - 2) % 3)
            wait_scatter((CH - 1) % 3)

        @pl.when(c == 0)
        def _lo():
            run(tlo_hbm)

        @pl.when(c == 1)
        def _hi():
            run(thi_hbm)

        plsc.subcore_barrier()
        pltpu.sync_copy(agg_sh.at[pl.ds(r0, ROWS_PER_TILE)],
                        out_hbm.at[c, pl.ds(r0, ROWS_PER_TILE)])

    return body


@functools.cache
def _sc_kernels():
    mesh = plsc.VectorSubcoreMesh(core_axis_name="c", subcore_axis_name="s")
    params = pltpu.CompilerParams(use_tc_tiling_on_sc=False)
    sc_degree = pl.kernel(
        _sc_degree_body,
        out_type=jax.ShapeDtypeStruct((2, N_PAD, 32), f32),
        mesh=mesh,
        compiler_params=params,
        scratch_types=(
            [pltpu.VMEM((K,), jnp.int32)] * 3
            + [pltpu.VMEM((K, 32), f32)]
            + [pltpu.VMEM_SHARED((N_PAD, 32), f32)]
            + [pltpu.SemaphoreType.DMA] * 6
        ),
    )
    sc_scatter = pl.kernel(
        _make_sc_scatter_body(),
        out_type=jax.ShapeDtypeStruct((2, N_PAD, 32), f32),
        mesh=mesh,
        compiler_params=params,
        scratch_types=(
            [pltpu.VMEM((K,), jnp.int32)] * 6
            + [pltpu.VMEM((K, 32), f32)] * 3
            + [pltpu.VMEM_SHARED((N_PAD, 32), f32)]
            + [pltpu.SemaphoreType.DMA] * 9
        ),
    )
    return sc_degree, sc_scatter


RB = 1024
RP = RB // 4
GRID = N_PAD // RB


def _scale1_body(d_ref, x_ref, wlo_ref, whi_ref, tlo_ref, thi_ref, dinv_ref):
    dinv = lax.rsqrt(d_ref[0] + d_ref[1] + 1.0)
    x4 = x_ref[...]
    tlo_ref[...] = dinv * jnp.dot(x4, wlo_ref[...], preferred_element_type=f32)
    thi_ref[...] = dinv * jnp.dot(x4, whi_ref[...], preferred_element_type=f32)
    dinv_ref[...] = dinv


_scale1 = pl.pallas_call(
    _scale1_body,
    grid=(GRID,),
    in_specs=[
        pl.BlockSpec((2, RP, 128), lambda i: (0, i, 0)),
        pl.BlockSpec((RP, 4 * D_IN), lambda i: (i, 0)),
        pl.BlockSpec((4 * D_IN, 128), lambda i: (0, 0)),
        pl.BlockSpec((4 * D_IN, 128), lambda i: (0, 0)),
    ],
    out_specs=[
        pl.BlockSpec((RP, 128), lambda i: (i, 0)),
        pl.BlockSpec((RP, 128), lambda i: (i, 0)),
        pl.BlockSpec((RP, 128), lambda i: (i, 0)),
    ],
    out_shape=[
        jax.ShapeDtypeStruct((N_PAD // 4, 128), f32),
        jax.ShapeDtypeStruct((N_PAD // 4, 128), f32),
        jax.ShapeDtypeStruct((N_PAD // 4, 128), f32),
    ],
)


def _mid_body(agg_ref, tlo_ref, thi_ref, dinv_ref,
              waa_ref, wba_ref, wab_ref, wbb_ref, blo_ref, bhi_ref,
              olo_ref, ohi_ref):
    dinv = dinv_ref[...]
    x2lo = jax.nn.relu(dinv * (agg_ref[0] + tlo_ref[...]) + blo_ref[...])
    x2hi = jax.nn.relu(dinv * (agg_ref[1] + thi_ref[...]) + bhi_ref[...])
    h2lo = (jnp.dot(x2lo, waa_ref[...], preferred_element_type=f32)
            + jnp.dot(x2hi, wba_ref[...], preferred_element_type=f32))
    h2hi = (jnp.dot(x2lo, wab_ref[...], preferred_element_type=f32)
            + jnp.dot(x2hi, wbb_ref[...], preferred_element_type=f32))
    olo_ref[...] = dinv * h2lo
    ohi_ref[...] = dinv * h2hi


_mid = pl.pallas_call(
    _mid_body,
    grid=(GRID,),
    in_specs=[
        pl.BlockSpec((2, RP, 128), lambda i: (0, i, 0)),
        pl.BlockSpec((RP, 128), lambda i: (i, 0)),
        pl.BlockSpec((RP, 128), lambda i: (i, 0)),
        pl.BlockSpec((RP, 128), lambda i: (i, 0)),
        pl.BlockSpec((128, 128), lambda i: (0, 0)),
        pl.BlockSpec((128, 128), lambda i: (0, 0)),
        pl.BlockSpec((128, 128), lambda i: (0, 0)),
        pl.BlockSpec((128, 128), lambda i: (0, 0)),
        pl.BlockSpec((1, 128), lambda i: (0, 0)),
        pl.BlockSpec((1, 128), lambda i: (0, 0)),
    ],
    out_specs=[
        pl.BlockSpec((RP, 128), lambda i: (i, 0)),
        pl.BlockSpec((RP, 128), lambda i: (i, 0)),
    ],
    out_shape=[
        jax.ShapeDtypeStruct((N_PAD // 4, 128), f32),
        jax.ShapeDtypeStruct((N_PAD // 4, 128), f32),
    ],
)


def _head_body(agg_ref, tlo_ref, thi_ref, dinv_ref,
               w3lo_ref, w3hi_ref, blo_ref, bhi_ref, s_ref, b3_ref, out_ref):
    dinv = dinv_ref[...]
    x3lo = jax.nn.relu(dinv * (agg_ref[0] + tlo_ref[...]) + blo_ref[...])
    x3hi = jax.nn.relu(dinv * (agg_ref[1] + thi_ref[...]) + bhi_ref[...])
    z = x3lo * w3lo_ref[...] + x3hi * w3hi_ref[...]
    out_ref[...] = (jnp.dot(z, s_ref[...], preferred_element_type=f32)
                    + b3_ref[0, 0])


_head = pl.pallas_call(
    _head_body,
    grid=(GRID,),
    in_specs=[
        pl.BlockSpec((2, RP, 128), lambda i: (0, i, 0)),
        pl.BlockSpec((RP, 128), lambda i: (i, 0)),
        pl.BlockSpec((RP, 128), lambda i: (i, 0)),
        pl.BlockSpec((RP, 128), lambda i: (i, 0)),
        pl.BlockSpec((1, 128), lambda i: (0, 0)),
        pl.BlockSpec((1, 128), lambda i: (0, 0)),
        pl.BlockSpec((1, 128), lambda i: (0, 0)),
        pl.BlockSpec((1, 128), lambda i: (0, 0)),
        pl.BlockSpec((128, 4), lambda i: (0, 0)),
        pl.BlockSpec((1, 8), lambda i: (0, 0)),
    ],
    out_specs=pl.BlockSpec((RP, 4), lambda i: (i, 0)),
    out_shape=jax.ShapeDtypeStruct((N_PAD // 4, 4), f32),
)


def kernel(obs, edge_index, W1, b1, W2, b2, W3, b3):
    src = edge_index[0]
    dst = edge_index[1]
    pad = E_PAD - E
    ar = jnp.arange(pad, dtype=jnp.int32)
    src_p = jnp.concatenate([src, ar % N])
    dst_p = jnp.concatenate([dst, N + ar % (N_PAD - N)])

    obs4 = jnp.pad(obs, ((0, N_PAD - N), (0, 0))).reshape(N_PAD // 4, 4 * D_IN)
    zeros32 = jnp.zeros((N_PAD, 32), f32)
    ones32 = jnp.ones((K, 32), f32)

    eye4 = jnp.eye(4, dtype=f32)
    w1lo = jnp.kron(eye4, W1[:, :32])
    w1hi = jnp.kron(eye4, W1[:, 32:])
    w2aa = jnp.kron(eye4, W2[:32, :32])
    w2ba = jnp.kron(eye4, W2[32:, :32])
    w2ab = jnp.kron(eye4, W2[:32, 32:])
    w2bb = jnp.kron(eye4, W2[32:, 32:])
    b1lo = jnp.tile(b1[:32], 4).reshape(1, 128)
    b1hi = jnp.tile(b1[32:], 4).reshape(1, 128)
    b2lo = jnp.tile(b2[:32], 4).reshape(1, 128)
    b2hi = jnp.tile(b2[32:], 4).reshape(1, 128)
    w3lo = jnp.tile(W3[:32, 0], 4).reshape(1, 128)
    w3hi = jnp.tile(W3[32:, 0], 4).reshape(1, 128)
    ssum = jnp.kron(eye4, jnp.ones((32, 1), f32))
    b3b = jnp.broadcast_to(b3.reshape(1, 1), (1, 8))

    _sc_degree, _sc_scatter = _sc_kernels()
    degp = _sc_degree(dst_p, ones32, zeros32)

    tab1lo, tab1hi, dinv = _scale1(degp.reshape(2, N_PAD // 4, 128),
                                   obs4, w1lo, w1hi)
    agg1 = _sc_scatter(tab1lo.reshape(N_PAD, 32), tab1hi.reshape(N_PAD, 32),
                       src_p, dst_p, zeros32)

    tab2lo, tab2hi = _mid(agg1.reshape(2, N_PAD // 4, 128), tab1lo, tab1hi,
                          dinv, w2aa, w2ba, w2ab, w2bb, b1lo, b1hi)
    agg2 = _sc_scatter(tab2lo.reshape(N_PAD, 32), tab2hi.reshape(N_PAD, 32),
                       src_p, dst_p, zeros32)

    y4 = _head(agg2.reshape(2, N_PAD // 4, 128), tab2lo, tab2hi, dinv,
               w3lo, w3hi, b2lo, b2hi, ssum, b3b)

    y = y4.reshape(-1)[:N]
    return y.reshape(-1, 15)[:, 3:].reshape(-1)


# speedup vs baseline: 1.3536x; 1.0620x over previous
import functools

import jax
import jax.numpy as jnp
from jax import lax
from jax.experimental import pallas as pl
from jax.experimental.pallas import tpu as pltpu
from jax.experimental.pallas import tpu_sc as plsc

N = 49995
E = 799920
D_IN = 128
D_H = 64

N_PAD = 50176
ROWS_PER_TILE = N_PAD // 16

K = 288
CH = 174
E_PAD = 16 * K * CH
DCH = E_PAD // (32 * K)

f32 = jnp.float32


def _sc_degree_body(dst_hbm, ones_hbm, zeros_hbm, out_hbm,
                    idx_v0, idx_v1, idx_v2, ones_v, deg_sh,
                    is0, is1, is2, ss0, ss1, ss2):
    c = lax.axis_index("c")
    s = lax.axis_index("s")
    pltpu.sync_copy(ones_hbm, ones_v)
    r0 = s * ROWS_PER_TILE
    pltpu.sync_copy(zeros_hbm.at[pl.ds(r0, ROWS_PER_TILE)],
                    deg_sh.at[pl.ds(r0, ROWS_PER_TILE)])
    plsc.subcore_barrier()
    base = (c * 16 + s) * DCH * K
    bufs = ((idx_v0, is0, ss0), (idx_v1, is1, ss1), (idx_v2, is2, ss2))

    def stage_idx(i, t):
        iv, isem, _ = bufs[t]
        pltpu.async_copy(dst_hbm.at[pl.ds(base + i * K, K)], iv, isem)

    def scatter(i, t):
        iv, isem, ss = bufs[t]
        pltpu.make_async_copy(dst_hbm.at[pl.ds(base + i * K, K)], iv,
                              isem).wait()
        pltpu.async_copy(ones_v, deg_sh.at[iv], ss, add=True)

    def wait_scatter(t):
        iv, _, ss = bufs[t]
        pltpu.make_async_copy(ones_v, deg_sh.at[iv], ss).wait()

    stage_idx(0, 0)
    stage_idx(1, 1)
    stage_idx(2, 2)
    scatter(0, 0)
    scatter(1, 1)

    peel = 3 + (DCH - 2) % 3
    loop_n = (DCH - 2 - peel) // 3

    @pl.loop(0, loop_n)
    def _outer(o):
        for u in range(3):
            i = 2 + 3 * o + u
            t = (2 + u) % 3
            wait_scatter(u % 3)
            stage_idx(i + 1, u % 3)
            scatter(i, t)

    for u in range(peel):
        i = 2 + 3 * loop_n + u
        t = (2 + u) % 3
        wait_scatter(u % 3)
        if i + 1 < DCH:
            stage_idx(i + 1, u % 3)
        scatter(i, t)

    wait_scatter((DCH - 2) % 3)
    wait_scatter((DCH - 1) % 3)
    plsc.subcore_barrier()
    pltpu.sync_copy(deg_sh.at[pl.ds(r0, ROWS_PER_TILE)],
                    out_hbm.at[c, pl.ds(r0, ROWS_PER_TILE)])


def _make_sc_scatter_body():
    def body(tlo_hbm, thi_hbm, src_hbm, dst_hbm, zeros_hbm, out_hbm,
             src_v0, dst_v0, src_v1, dst_v1, src_v2, dst_v2,
             rows_v0, rows_v1, rows_v2, agg_sh,
             is0, is1, is2, gs0, gs1, gs2, ss0, ss1, ss2):
        c = lax.axis_index("c")
        s = lax.axis_index("s")
        r0 = s * ROWS_PER_TILE
        pltpu.sync_copy(zeros_hbm.at[pl.ds(r0, ROWS_PER_TILE)],
                        agg_sh.at[pl.ds(r0, ROWS_PER_TILE)])
        plsc.subcore_barrier()
        base = s * CH * K
        bufs = ((src_v0, dst_v0, rows_v0, is0, gs0, ss0),
                (src_v1, dst_v1, rows_v1, is1, gs1, ss1),
                (src_v2, dst_v2, rows_v2, is2, gs2, ss2))

        def stage_idx(i, t):
            sv, dv, rv, isem, gsem, ssem = bufs[t]
            pltpu.async_copy(src_hbm.at[pl.ds(base + i * K, K)], sv, isem)
            pltpu.async_copy(dst_hbm.at[pl.ds(base + i * K, K)], dv, isem)

        def wait_idx(i, t):
            sv, dv, rv, isem, gsem, ssem = bufs[t]
            pltpu.make_async_copy(src_hbm.at[pl.ds(base + i * K, K)], sv,
                                  isem).wait()
            pltpu.make_async_copy(dst_hbm.at[pl.ds(base + i * K, K)], dv,
                                  isem).wait()

        def run(tab):
            def start_gather(i, t):
                sv, dv, rv, isem, gsem, ssem = bufs[t]
                wait_idx(i, t)
                pltpu.async_copy(tab.at[sv], rv, gsem)

            def finish_scatter(t):
                sv, dv, rv, isem, gsem, ssem = bufs[t]
                pltpu.make_async_copy(tab.at[sv], rv, gsem).wait()
                pltpu.async_copy(rv, agg_sh.at[dv], ssem, add=True)

            def wait_scatter(t):
                sv, dv, rv, isem, gsem, ssem = bufs[t]
                pltpu.make_async_copy(rv, agg_sh.at[dv], ssem).wait()

            stage_idx(0, 0)
            stage_idx(1, 1)
            stage_idx(2, 2)
            start_gather(0, 0)
            start_gather(1, 1)
            finish_scatter(0)
            wait_scatter(0)
            stage_idx(3, 0)
            start_gather(2, 2)
            finish_scatter(1)

            @pl.loop(0, (CH - 6) // 3)
            def _outer(o):
                for u in range(3):
                    i = 3 + 3 * o + u
                    t = u
                    wait_scatter((u + 1) % 3)
                    stage_idx(i + 1, (u + 1) % 3)
                    start_gather(i, t)
                    finish_scatter((u + 2) % 3)

            for i in range(CH - 3, CH):
                t = i % 3
                wait_scatter((i + 1) % 3)
                if i + 1 < CH:
                    stage_idx(i + 1, (i + 1) % 3)
                start_gather(i, t)
                finish_scatter((i + 2) % 3)

            finish_scatter((CH - 1) % 3)
            wait_scatter((CH - 2) % 3)
            wait_scatter((CH - 1) % 3)

        @pl.when(c == 0)
        def _lo():
            run(tlo_hbm)

        @pl.when(c == 1)
        def _hi():
            run(thi_hbm)

        plsc.subcore_barrier()
        pltpu.sync_copy(agg_sh.at[pl.ds(r0, ROWS_PER_TILE)],
                        out_hbm.at[c, pl.ds(r0, ROWS_PER_TILE)])

    return body


@functools.cache
def _sc_kernels():
    mesh = plsc.VectorSubcoreMesh(core_axis_name="c", subcore_axis_name="s")
    params = pltpu.CompilerParams(use_tc_tiling_on_sc=False)
    sc_degree = pl.kernel(
        _sc_degree_body,
        out_type=jax.ShapeDtypeStruct((2, N_PAD, 32), f32),
        mesh=mesh,
        compiler_params=params,
        scratch_types=(
            [pltpu.VMEM((K,), jnp.int32)] * 3
            + [pltpu.VMEM((K, 32), f32)]
            + [pltpu.VMEM_SHARED((N_PAD, 32), f32)]
            + [pltpu.SemaphoreType.DMA] * 6
        ),
    )
    sc_scatter = pl.kernel(
        _make_sc_scatter_body(),
        out_type=jax.ShapeDtypeStruct((2, N_PAD, 32), f32),
        mesh=mesh,
        compiler_params=params,
        scratch_types=(
            [pltpu.VMEM((K,), jnp.int32)] * 6
            + [pltpu.VMEM((K, 32), f32)] * 3
            + [pltpu.VMEM_SHARED((N_PAD, 32), f32)]
            + [pltpu.SemaphoreType.DMA] * 9
        ),
    )
    return sc_degree, sc_scatter


RB = 1792
RP = RB // 4
GRID = N_PAD // RB


def _scale1_body(d_ref, x_ref, wlo_ref, whi_ref, tlo_ref, thi_ref, dinv_ref):
    dinv = lax.rsqrt(d_ref[0] + d_ref[1] + 1.0)
    x4 = x_ref[...]
    tlo_ref[...] = dinv * jnp.dot(x4, wlo_ref[...], preferred_element_type=f32)
    thi_ref[...] = dinv * jnp.dot(x4, whi_ref[...], preferred_element_type=f32)
    dinv_ref[...] = dinv


_scale1 = pl.pallas_call(
    _scale1_body,
    grid=(GRID,),
    in_specs=[
        pl.BlockSpec((2, RP, 128), lambda i: (0, i, 0)),
        pl.BlockSpec((RP, 4 * D_IN), lambda i: (i, 0)),
        pl.BlockSpec((4 * D_IN, 128), lambda i: (0, 0)),
        pl.BlockSpec((4 * D_IN, 128), lambda i: (0, 0)),
    ],
    out_specs=[
        pl.BlockSpec((RP, 128), lambda i: (i, 0)),
        pl.BlockSpec((RP, 128), lambda i: (i, 0)),
        pl.BlockSpec((RP, 128), lambda i: (i, 0)),
    ],
    out_shape=[
        jax.ShapeDtypeStruct((N_PAD // 4, 128), f32),
        jax.ShapeDtypeStruct((N_PAD // 4, 128), f32),
        jax.ShapeDtypeStruct((N_PAD // 4, 128), f32),
    ],
)


def _mid_body(agg_ref, tlo_ref, thi_ref, dinv_ref,
              waa_ref, wba_ref, wab_ref, wbb_ref, blo_ref, bhi_ref,
              olo_ref, ohi_ref):
    dinv = dinv_ref[...]
    x2lo = jax.nn.relu(dinv * (agg_ref[0] + tlo_ref[...]) + blo_ref[...])
    x2hi = jax.nn.relu(dinv * (agg_ref[1] + thi_ref[...]) + bhi_ref[...])
    h2lo = (jnp.dot(x2lo, waa_ref[...], preferred_element_type=f32)
            + jnp.dot(x2hi, wba_ref[...], preferred_element_type=f32))
    h2hi = (jnp.dot(x2lo, wab_ref[...], preferred_element_type=f32)
            + jnp.dot(x2hi, wbb_ref[...], preferred_element_type=f32))
    olo_ref[...] = dinv * h2lo
    ohi_ref[...] = dinv * h2hi


_mid = pl.pallas_call(
    _mid_body,
    grid=(GRID,),
    in_specs=[
        pl.BlockSpec((2, RP, 128), lambda i: (0, i, 0)),
        pl.BlockSpec((RP, 128), lambda i: (i, 0)),
        pl.BlockSpec((RP, 128), lambda i: (i, 0)),
        pl.BlockSpec((RP, 128), lambda i: (i, 0)),
        pl.BlockSpec((128, 128), lambda i: (0, 0)),
        pl.BlockSpec((128, 128), lambda i: (0, 0)),
        pl.BlockSpec((128, 128), lambda i: (0, 0)),
        pl.BlockSpec((128, 128), lambda i: (0, 0)),
        pl.BlockSpec((1, 128), lambda i: (0, 0)),
        pl.BlockSpec((1, 128), lambda i: (0, 0)),
    ],
    out_specs=[
        pl.BlockSpec((RP, 128), lambda i: (i, 0)),
        pl.BlockSpec((RP, 128), lambda i: (i, 0)),
    ],
    out_shape=[
        jax.ShapeDtypeStruct((N_PAD // 4, 128), f32),
        jax.ShapeDtypeStruct((N_PAD // 4, 128), f32),
    ],
)


def _head_body(agg_ref, tlo_ref, thi_ref, dinv_ref,
               w3lo_ref, w3hi_ref, blo_ref, bhi_ref, s_ref, b3_ref, out_ref):
    dinv = dinv_ref[...]
    x3lo = jax.nn.relu(dinv * (agg_ref[0] + tlo_ref[...]) + blo_ref[...])
    x3hi = jax.nn.relu(dinv * (agg_ref[1] + thi_ref[...]) + bhi_ref[...])
    z = x3lo * w3lo_ref[...] + x3hi * w3hi_ref[...]
    out_ref[...] = (jnp.dot(z, s_ref[...], preferred_element_type=f32)
                    + b3_ref[0, 0])


_head = pl.pallas_call(
    _head_body,
    grid=(GRID,),
    in_specs=[
        pl.BlockSpec((2, RP, 128), lambda i: (0, i, 0)),
        pl.BlockSpec((RP, 128), lambda i: (i, 0)),
        pl.BlockSpec((RP, 128), lambda i: (i, 0)),
        pl.BlockSpec((RP, 128), lambda i: (i, 0)),
        pl.BlockSpec((1, 128), lambda i: (0, 0)),
        pl.BlockSpec((1, 128), lambda i: (0, 0)),
        pl.BlockSpec((1, 128), lambda i: (0, 0)),
        pl.BlockSpec((1, 128), lambda i: (0, 0)),
        pl.BlockSpec((128, 4), lambda i: (0, 0)),
        pl.BlockSpec((1, 8), lambda i: (0, 0)),
    ],
    out_specs=pl.BlockSpec((RP, 4), lambda i: (i, 0)),
    out_shape=jax.ShapeDtypeStruct((N_PAD // 4, 4), f32),
)


def kernel(obs, edge_index, W1, b1, W2, b2, W3, b3):
    src = edge_index[0]
    dst = edge_index[1]
    pad = E_PAD - E
    ar = jnp.arange(pad, dtype=jnp.int32)
    src_p = jnp.concatenate([src, ar % N])
    dst_p = jnp.concatenate([dst, N + ar % (N_PAD - N)])

    obs4 = jnp.pad(obs, ((0, N_PAD - N), (0, 0))).reshape(N_PAD // 4, 4 * D_IN)
    zeros32 = jnp.zeros((N_PAD, 32), f32)
    ones32 = jnp.ones((K, 32), f32)

    eye4 = jnp.eye(4, dtype=f32)
    w1lo = jnp.kron(eye4, W1[:, :32])
    w1hi = jnp.kron(eye4, W1[:, 32:])
    w2aa = jnp.kron(eye4, W2[:32, :32])
    w2ba = jnp.kron(eye4, W2[32:, :32])
    w2ab = jnp.kron(eye4, W2[:32, 32:])
    w2bb = jnp.kron(eye4, W2[32:, 32:])
    b1lo = jnp.tile(b1[:32], 4).reshape(1, 128)
    b1hi = jnp.tile(b1[32:], 4).reshape(1, 128)
    b2lo = jnp.tile(b2[:32], 4).reshape(1, 128)
    b2hi = jnp.tile(b2[32:], 4).reshape(1, 128)
    w3lo = jnp.tile(W3[:32, 0], 4).reshape(1, 128)
    w3hi = jnp.tile(W3[32:, 0], 4).reshape(1, 128)
    ssum = jnp.kron(eye4, jnp.ones((32, 1), f32))
    b3b = jnp.broadcast_to(b3.reshape(1, 1), (1, 8))

    _sc_degree, _sc_scatter = _sc_kernels()
    degp = _sc_degree(dst_p, ones32, zeros32)

    tab1lo, tab1hi, dinv = _scale1(degp.reshape(2, N_PAD // 4, 128),
                                   obs4, w1lo, w1hi)
    agg1 = _sc_scatter(tab1lo.reshape(N_PAD, 32), tab1hi.reshape(N_PAD, 32),
                       src_p, dst_p, zeros32)

    tab2lo, tab2hi = _mid(agg1.reshape(2, N_PAD // 4, 128), tab1lo, tab1hi,
                          dinv, w2aa, w2ba, w2ab, w2bb, b1lo, b1hi)
    agg2 = _sc_scatter(tab2lo.reshape(N_PAD, 32), tab2hi.reshape(N_PAD, 32),
                       src_p, dst_p, zeros32)

    y4 = _head(agg2.reshape(2, N_PAD // 4, 128), tab2lo, tab2hi, dinv,
               w3lo, w3hi, b2lo, b2hi, ssum, b3b)

    y = y4.reshape(-1)[:N]
    return y.reshape(-1, 15)[:, 3:].reshape(-1)
